# baseline (speedup 1.0000x reference)
import functools

import jax
import jax.numpy as jnp
from jax import lax
from jax.experimental import pallas as pl
from jax.experimental.pallas import tpu as pltpu
from jax.experimental.pallas import tpu_sc as plsc

_B, _Q, _C, _L = 16, 20000, 91, 20
_LPAD = 32
_NV = 6
_OFFS = (0, 16, 32, 48, 64, 75)
_HALF = _Q // 2
_CH = 400
_NCH = _HALF // _CH
_CW = _CH * _C
_UNROLL = 4


def _row_update(bref, base_off, maxs, idxs, rid):
    new_maxs, new_idxs = [], []
    for k in range(_NV):
        v = bref[pl.ds(base_off + _OFFS[k], 16)]
        gt = v > maxs[k]
        new_maxs.append(jnp.where(gt, v, maxs[k]))
        new_idxs.append(jnp.where(gt, rid, idxs[k]))
    return new_maxs, new_idxs, rid + 1


@functools.partial(
    pl.kernel,
    out_type=[
        jax.ShapeDtypeStruct((_B, _LPAD), jnp.float32),
        jax.ShapeDtypeStruct((_B, 128), jnp.float32),
    ],
    mesh=plsc.VectorSubcoreMesh(core_axis_name="c", subcore_axis_name="s"),
    scratch_types=[
        pltpu.VMEM((_CW,), jnp.float32),
        pltpu.VMEM((_CW,), jnp.float32),
        pltpu.VMEM((128,), jnp.float32),
        pltpu.VMEM((128,), jnp.int32),
        pltpu.VMEM((128,), jnp.float32),
        pltpu.VMEM((128,), jnp.int32),
        pltpu.VMEM((_LPAD,), jnp.int32),
        pltpu.VMEM((_LPAD,), jnp.float32),
        pltpu.VMEM((_L * 128 + 16,), jnp.float32),
        pltpu.VMEM((128,), jnp.float32),
        pltpu.VMEM_SHARED((16 * 128,), jnp.float32),
        pltpu.VMEM_SHARED((16 * 128,), jnp.int32),
        pltpu.SemaphoreType.DMA,
        pltpu.SemaphoreType.DMA,
        pltpu.SemaphoreType.DMA,
    ],
)
def _sc_post(lg_hbm, bx_hbm, lab_hbm, sc_out, bx_out,
             buf_a, buf_b, maxb, idxb, pmaxb, pidxb, labb, scb, boxg, boxb,
             sh_max, sh_idx, sem_a, sem_b, sem_g):
    c = lax.axis_index("c")
    s = lax.axis_index("s")
    b = c * 8 + s // 2
    h = s % 2

    base_word = b * (_Q * _C) + h * (_HALF * _C)
    bufs = (buf_a, buf_b)
    sems = (sem_a, sem_b)

    def start(i):
        return pltpu.async_copy(
            lg_hbm.at[pl.ds(base_word + i * _CW, _CW)],
            bufs[i % 2],
            sems[i % 2],
        )

    maxs = [jnp.full((16,), -jnp.inf, jnp.float32) for _ in range(_NV)]
    idxs = [jnp.zeros((16,), jnp.int32) for _ in range(_NV)]

    cps = [start(0), None]
    for i in range(_NCH):
        if i + 1 < _NCH:
            cps[(i + 1) % 2] = start(i + 1)
        cps[i % 2].wait()
        bref = bufs[i % 2]
        rid0 = jnp.full((16,), h * _HALF + i * _CH, jnp.int32)

        def chunk_body(j, carry, bref=bref):
            ms, ids, rid = carry[:_NV], carry[_NV:2 * _NV], carry[2 * _NV]
            off = j * (_UNROLL * _C)
            for u in range(_UNROLL):
                ms, ids, rid = _row_update(bref, off + u * _C, ms, ids, rid)
            return tuple(ms) + tuple(ids) + (rid,)

        carry = lax.fori_loop(0, _CH // _UNROLL, chunk_body,
                              tuple(maxs) + tuple(idxs) + (rid0,))
        maxs, idxs = list(carry[:_NV]), list(carry[_NV:2 * _NV])

    for k in range(_NV):
        maxb[pl.ds(16 * k, 16)] = maxs[k]
        idxb[pl.ds(16 * k, 16)] = idxs[k]
    pltpu.sync_copy(maxb, sh_max.at[pl.ds(s * 128, 128)])
    pltpu.sync_copy(idxb, sh_idx.at[pl.ds(s * 128, 128)])
    plsc.subcore_barrier()

    @pl.when(h == 0)
    def _finish():
        pltpu.sync_copy(sh_max.at[pl.ds((s + 1) * 128, 128)], pmaxb)
        pltpu.sync_copy(sh_idx.at[pl.ds((s + 1) * 128, 128)], pidxb)
        for k in range(_NV):
            pm = pmaxb[pl.ds(16 * k, 16)]
            pi = pidxb[pl.ds(16 * k, 16)]
            gt = pm > maxs[k]
            maxb[pl.ds(16 * k, 16)] = jnp.where(gt, pm, maxs[k])
            idxb[pl.ds(16 * k, 16)] = jnp.where(gt, pi, idxs[k])

        pltpu.sync_copy(lab_hbm.at[b], labb)
        lanes = lax.iota(jnp.int32, 16)
        n_words = _B * _Q * 4
        sc_acc = [jnp.zeros((16,), jnp.float32) for _ in range(2)]
        box_cps, box_offs = [], []
        for t in range(_L):
            t0 = 16 if t >= 16 else 0
            lab = labb[pl.ds(t0, 16)][t - t0]
            g = jnp.where(lab >= 80, lab + 5, lab)
            mv = maxb[pl.ds(g, 16)][0]
            iv = idxb[pl.ds(g, 16)][0]
            m = lanes == (t % 16)
            sc_acc[t // 16] = jnp.where(m, mv, sc_acc[t // 16])
            a4 = (iv + b * _Q) * 4
            start = pl.multiple_of(jnp.minimum(a4 - a4 % 8, n_words - 128), 8)
            box_offs.append(a4 - start)
            box_cps.append(pltpu.async_copy(
                bx_hbm.at[pl.ds(start, 128)],
                boxg.at[pl.ds(t * 128, 128)],
                sem_g,
            ))
        for t in range(_LPAD // 16):
            scb[pl.ds(16 * t, 16)] = 1.0 / (1.0 + jnp.exp(-sc_acc[t]))
        pltpu.sync_copy(scb, sc_out.at[b])
        box_acc = [jnp.zeros((16,), jnp.float32) for _ in range(8)]
        for t in range(_L):
            box_cps[t].wait()
            r = (t % 4) * 4
            v = boxg[pl.ds(t * 128 + box_offs[t] - r, 16)]
            m4 = (lanes >= r) & (lanes < r + 4)
            box_acc[t // 4] = jnp.where(m4, v, box_acc[t // 4])
        for w in range(8):
            boxb[pl.ds(16 * w, 16)] = box_acc[w]
        pltpu.sync_copy(boxb, bx_out.at[b])


def kernel(pred_logits, pred_boxes, target_sizes, target_labels):
    del target_sizes
    lg = pred_logits.reshape(_B * _Q * _C)
    bx = pred_boxes.reshape(_B * _Q * 4)
    lab = jnp.pad(target_labels, ((0, 0), (0, _LPAD - _L)))
    scores, boxes = _sc_post(lg, bx, lab)
    return scores[:, :_L], target_labels, boxes.reshape(_B, 32, 4)[:, :_L]


# speedup vs baseline: 1.2810x; 1.2810x over previous
import functools

import jax
import jax.numpy as jnp
from jax import lax
from jax.experimental import pallas as pl
from jax.experimental.pallas import tpu as pltpu
from jax.experimental.pallas import tpu_sc as plsc

_B, _Q, _C, _L = 16, 20000, 91, 20
_LPAD = 32
_CPAD = 112
_CHQ = 2000
_SL = 8


def _tc_body(x_ref, mx_ref, ix_ref):
    qi = pl.program_id(1)

    sub = lax.broadcasted_iota(jnp.int32, (_SL, _C), 0)

    def fold(j, carry):
        acc_m, acc_i = carry
        v = x_ref[0, pl.ds(j * _SL, _SL), :]
        gt = v > acc_m
        return jnp.where(gt, v, acc_m), jnp.where(gt, j, acc_i)

    acc_m = jnp.full((_SL, _C), -jnp.inf, jnp.float32)
    acc_i = jnp.zeros((_SL, _C), jnp.int32)
    acc_m, acc_i = lax.fori_loop(0, _CHQ // _SL, fold, (acc_m, acc_i))

    rows = acc_i * _SL + sub + qi * _CHQ
    m = jnp.max(acc_m, axis=0)
    idx = jnp.min(jnp.where(acc_m == m[None, :], rows, _B * _Q), axis=0)

    mpad = jnp.concatenate(
        [m, jnp.full((_CPAD - _C,), -jnp.inf, jnp.float32)])
    ipad = jnp.concatenate([idx, jnp.zeros((_CPAD - _C,), jnp.int32)])

    @pl.when(qi == 0)
    def _init():
        mx_ref[0, 0, :] = mpad
        ix_ref[0, 0, :] = ipad

    @pl.when(qi > 0)
    def _merge():
        old_m = mx_ref[0, 0, :]
        old_i = ix_ref[0, 0, :]
        better = mpad > old_m
        mx_ref[0, 0, :] = jnp.where(better, mpad, old_m)
        ix_ref[0, 0, :] = jnp.where(better, ipad, old_i)


_tc_reduce = pl.pallas_call(
    _tc_body,
    grid=(_B, _Q // _CHQ),
    in_specs=[pl.BlockSpec((1, _CHQ, _C), lambda b, q: (b, q, 0))],
    out_specs=[
        pl.BlockSpec((1, 1, _CPAD), lambda b, q: (b, 0, 0)),
        pl.BlockSpec((1, 1, _CPAD), lambda b, q: (b, 0, 0)),
    ],
    out_shape=[
        jax.ShapeDtypeStruct((_B, 1, _CPAD), jnp.float32),
        jax.ShapeDtypeStruct((_B, 1, _CPAD), jnp.int32),
    ],
)


@functools.partial(
    pl.kernel,
    out_type=[
        jax.ShapeDtypeStruct((_B, _LPAD), jnp.float32),
        jax.ShapeDtypeStruct((_B, 128), jnp.float32),
    ],
    mesh=plsc.VectorSubcoreMesh(core_axis_name="c", subcore_axis_name="s"),
    scratch_types=[
        pltpu.VMEM((_CPAD,), jnp.float32),
        pltpu.VMEM((_CPAD,), jnp.int32),
        pltpu.VMEM((_LPAD,), jnp.int32),
        pltpu.VMEM((_LPAD,), jnp.float32),
        pltpu.VMEM((_L * 128 + 16,), jnp.float32),
        pltpu.VMEM((128,), jnp.float32),
        pltpu.SemaphoreType.DMA,
    ],
)
def _sc_finish(mx_hbm, ix_hbm, bx_hbm, lab_hbm, sc_out, bx_out,
               maxb, idxb, labb, scb, boxg, boxb, sem_g):
    c = lax.axis_index("c")
    s = lax.axis_index("s")

    @pl.when(c == 0)
    def _work():
        b = s
        pltpu.sync_copy(mx_hbm.at[b], maxb)
        pltpu.sync_copy(ix_hbm.at[b], idxb)
        pltpu.sync_copy(lab_hbm.at[b], labb)
        lanes = lax.iota(jnp.int32, 16)
        n_words = _B * _Q * 4
        sc_acc = [jnp.zeros((16,), jnp.float32) for _ in range(2)]
        box_cps, box_offs = [], []
        for t in range(_L):
            t0 = 16 if t >= 16 else 0
            lab = labb[pl.ds(t0, 16)][t - t0]
            mv = maxb[pl.ds(lab, 16)][0]
            iv = idxb[pl.ds(lab, 16)][0]
            m = lanes == (t % 16)
            sc_acc[t // 16] = jnp.where(m, mv, sc_acc[t // 16])
            a4 = (iv + b * _Q) * 4
            start = pl.multiple_of(jnp.minimum(a4 - a4 % 8, n_words - 128), 8)
            box_offs.append(a4 - start)
            box_cps.append(pltpu.async_copy(
                bx_hbm.at[pl.ds(start, 128)],
                boxg.at[pl.ds(t * 128, 128)],
                sem_g,
            ))
        for t in range(_LPAD // 16):
            scb[pl.ds(16 * t, 16)] = 1.0 / (1.0 + jnp.exp(-sc_acc[t]))
        pltpu.sync_copy(scb, sc_out.at[b])
        box_acc = [jnp.zeros((16,), jnp.float32) for _ in range(8)]
        for t in range(_L):
            box_cps[t].wait()
            r = (t % 4) * 4
            v = boxg[pl.ds(t * 128 + box_offs[t] - r, 16)]
            m4 = (lanes >= r) & (lanes < r + 4)
            box_acc[t // 4] = jnp.where(m4, v, box_acc[t // 4])
        for w in range(8):
            boxb[pl.ds(16 * w, 16)] = box_acc[w]
        pltpu.sync_copy(boxb, bx_out.at[b])


def kernel(pred_logits, pred_boxes, target_sizes, target_labels):
    del target_sizes
    mx, ix = _tc_reduce(pred_logits)
    bx = pred_boxes.reshape(_B * _Q * 4)
    lab = jnp.pad(target_labels, ((0, 0), (0, _LPAD - _L)))
    scores, boxes = _sc_finish(mx.reshape(_B, _CPAD), ix.reshape(_B, _CPAD),
                               bx, lab)
    return scores[:, :_L], target_labels, boxes.reshape(_B, 32, 4)[:, :_L]


# speedup vs baseline: 1.5650x; 1.2217x over previous
import functools

import jax
import jax.numpy as jnp
from jax import lax
from jax.experimental import pallas as pl
from jax.experimental.pallas import tpu as pltpu
from jax.experimental.pallas import tpu_sc as plsc

_B, _Q, _C, _L = 16, 20000, 91, 20
_LPAD = 32
_CPAD = 112
_CHQ = 2000
_SL = 8


def _tc_body(x_ref, mx_ref, ix_ref):
    qi = pl.program_id(1)

    sub = lax.broadcasted_iota(jnp.int32, (_SL, _C), 0)

    acc_m = x_ref[0, pl.ds(0, _SL), :]
    acc_i = jnp.zeros((_SL, _C), jnp.int32)
    for j in range(1, _CHQ // _SL):
        v = x_ref[0, pl.ds(j * _SL, _SL), :]
        gt = v > acc_m
        acc_m = jnp.where(gt, v, acc_m)
        acc_i = jnp.where(gt, j, acc_i)

    rows = acc_i * _SL + sub + qi * _CHQ
    m = jnp.max(acc_m, axis=0)
    idx = jnp.min(jnp.where(acc_m == m[None, :], rows, _B * _Q), axis=0)

    mpad = jnp.concatenate(
        [m, jnp.full((_CPAD - _C,), -jnp.inf, jnp.float32)])
    ipad = jnp.concatenate([idx, jnp.zeros((_CPAD - _C,), jnp.int32)])

    @pl.when(qi == 0)
    def _init():
        mx_ref[0, 0, :] = mpad
        ix_ref[0, 0, :] = ipad

    @pl.when(qi > 0)
    def _merge():
        old_m = mx_ref[0, 0, :]
        old_i = ix_ref[0, 0, :]
        better = mpad > old_m
        mx_ref[0, 0, :] = jnp.where(better, mpad, old_m)
        ix_ref[0, 0, :] = jnp.where(better, ipad, old_i)


_tc_reduce = pl.pallas_call(
    _tc_body,
    grid=(_B, _Q // _CHQ),
    in_specs=[pl.BlockSpec((1, _CHQ, _C), lambda b, q: (b, q, 0))],
    out_specs=[
        pl.BlockSpec((1, 1, _CPAD), lambda b, q: (b, 0, 0)),
        pl.BlockSpec((1, 1, _CPAD), lambda b, q: (b, 0, 0)),
    ],
    out_shape=[
        jax.ShapeDtypeStruct((_B, 1, _CPAD), jnp.float32),
        jax.ShapeDtypeStruct((_B, 1, _CPAD), jnp.int32),
    ],
)


@functools.partial(
    pl.kernel,
    out_type=[
        jax.ShapeDtypeStruct((_B, _LPAD), jnp.float32),
        jax.ShapeDtypeStruct((_B, 128), jnp.float32),
    ],
    mesh=plsc.VectorSubcoreMesh(core_axis_name="c", subcore_axis_name="s"),
    scratch_types=[
        pltpu.VMEM((_CPAD,), jnp.float32),
        pltpu.VMEM((_CPAD,), jnp.int32),
        pltpu.VMEM((_LPAD,), jnp.int32),
        pltpu.VMEM((_LPAD,), jnp.float32),
        pltpu.VMEM((_L * 128 + 16,), jnp.float32),
        pltpu.VMEM((128,), jnp.float32),
        pltpu.SemaphoreType.DMA,
    ],
)
def _sc_finish(mx_hbm, ix_hbm, bx_hbm, lab_hbm, sc_out, bx_out,
               maxb, idxb, labb, scb, boxg, boxb, sem_g):
    c = lax.axis_index("c")
    s = lax.axis_index("s")

    @pl.when(c == 0)
    def _work():
        b = s
        pltpu.sync_copy(mx_hbm.at[b], maxb)
        pltpu.sync_copy(ix_hbm.at[b], idxb)
        pltpu.sync_copy(lab_hbm.at[b], labb)
        lanes = lax.iota(jnp.int32, 16)
        n_words = _B * _Q * 4
        sc_acc = [jnp.zeros((16,), jnp.float32) for _ in range(2)]
        box_cps, box_offs = [], []
        for t in range(_L):
            t0 = 16 if t >= 16 else 0
            lab = labb[pl.ds(t0, 16)][t - t0]
            mv = maxb[pl.ds(lab, 16)][0]
            iv = idxb[pl.ds(lab, 16)][0]
            m = lanes == (t % 16)
            sc_acc[t // 16] = jnp.where(m, mv, sc_acc[t // 16])
            a4 = (iv + b * _Q) * 4
            start = pl.multiple_of(jnp.minimum(a4 - a4 % 8, n_words - 128), 8)
            box_offs.append(a4 - start)
            box_cps.append(pltpu.async_copy(
                bx_hbm.at[pl.ds(start, 128)],
                boxg.at[pl.ds(t * 128, 128)],
                sem_g,
            ))
        for t in range(_LPAD // 16):
            scb[pl.ds(16 * t, 16)] = 1.0 / (1.0 + jnp.exp(-sc_acc[t]))
        pltpu.sync_copy(scb, sc_out.at[b])
        box_acc = [jnp.zeros((16,), jnp.float32) for _ in range(8)]
        for t in range(_L):
            box_cps[t].wait()
            r = (t % 4) * 4
            v = boxg[pl.ds(t * 128 + box_offs[t] - r, 16)]
            m4 = (lanes >= r) & (lanes < r + 4)
            box_acc[t // 4] = jnp.where(m4, v, box_acc[t // 4])
        for w in range(8):
            boxb[pl.ds(16 * w, 16)] = box_acc[w]
        pltpu.sync_copy(boxb, bx_out.at[b])


def kernel(pred_logits, pred_boxes, target_sizes, target_labels):
    del target_sizes
    mx, ix = _tc_reduce(pred_logits)
    bx = pred_boxes.reshape(_B * _Q * 4)
    lab = jnp.pad(target_labels, ((0, 0), (0, _LPAD - _L)))
    scores, boxes = _sc_finish(mx.reshape(_B, _CPAD), ix.reshape(_B, _CPAD),
                               bx, lab)
    return scores[:, :_L], target_labels, boxes.reshape(_B, 32, 4)[:, :_L]
